# Initial kernel scaffold; baseline (speedup 1.0000x reference)
import jax
import jax.numpy as jnp
from jax.experimental import pallas as pl


def kernel(X, emb, W1, b1, W2, b2):
    raise NotImplementedError("write your pallas kernel here")


# speedup vs baseline: 93.7230x; 93.7230x over previous
import jax
import jax.numpy as jnp
from jax.experimental import pallas as pl
from jax.experimental.pallas import tpu as pltpu

_IN_DIM = 100
_N_BINS = 10
_EMB = 16
_HID = 64
_OUT = 64
_FPAD = 128
_NROWS = _N_BINS * _FPAD


def _colmax_kernel(x_ref, o_ref):
    i = pl.program_id(0)
    part = jnp.max(jnp.abs(x_ref[...]), axis=0, keepdims=True)

    @pl.when(i == 0)
    def _():
        o_ref[...] = part

    @pl.when(i > 0)
    def _():
        o_ref[...] = jnp.maximum(o_ref[...], part)


def _fold_kernel(e2_ref, w1t_ref, t2_ref):
    t2 = jax.lax.dot(e2_ref[...], w1t_ref[...],
                     preferred_element_type=jnp.float32)
    t2_ref[...] = t2.astype(jnp.bfloat16)


def _main_kernel(x_ref, d_ref, t2_ref, b1_ref, w2t_ref, b2_ref, o_ref,
                 onehot_scr):
    x = x_ref[...]
    d = d_ref[...]
    bins = jnp.clip(x / d * (_N_BINS / 2.0) + _N_BINS / 2.0,
                    0.0, _N_BINS - 1).astype(jnp.int32)
    pad = jnp.full((x.shape[0], _FPAD - _IN_DIM), -1, jnp.int32)
    binp = jnp.concatenate([bins, pad], axis=1)
    for n in range(_N_BINS):
        onehot_scr[:, n * _FPAD:(n + 1) * _FPAD] = (
            (binp == n).astype(jnp.bfloat16))
    h = jax.lax.dot(onehot_scr[...], t2_ref[...],
                    preferred_element_type=jnp.float32)
    h = jnp.maximum(h + b1_ref[...], 0.0)
    out = jax.lax.dot(h, w2t_ref[...], preferred_element_type=jnp.float32)
    o_ref[...] = out + b2_ref[...]


def kernel(X, emb, W1, b1, W2, b2):
    B, IN = X.shape

    G1 = 8
    colmax = pl.pallas_call(
        _colmax_kernel,
        grid=(G1,),
        in_specs=[pl.BlockSpec((B // G1, IN), lambda i: (i, 0))],
        out_specs=pl.BlockSpec((1, IN), lambda i: (0, 0)),
        out_shape=jax.ShapeDtypeStruct((1, IN), jnp.float32),
    )(X)

    embT = jnp.transpose(emb, (1, 0, 2))
    z = jnp.zeros((_N_BINS, _FPAD, _IN_DIM, _EMB), jnp.float32)
    f_idx = jnp.arange(_IN_DIM)
    E2 = z.at[:, f_idx, f_idx, :].set(embT).reshape(_NROWS, _IN_DIM * _EMB)
    T2 = pl.pallas_call(
        _fold_kernel,
        out_shape=jax.ShapeDtypeStruct((_NROWS, _HID), jnp.bfloat16),
    )(E2, W1.T)

    BT = 1024
    G = B // BT
    out = pl.pallas_call(
        _main_kernel,
        grid=(G,),
        in_specs=[
            pl.BlockSpec((BT, IN), lambda i: (i, 0)),
            pl.BlockSpec((1, IN), lambda i: (0, 0)),
            pl.BlockSpec((_NROWS, _HID), lambda i: (0, 0)),
            pl.BlockSpec((1, _HID), lambda i: (0, 0)),
            pl.BlockSpec((_HID, _OUT), lambda i: (0, 0)),
            pl.BlockSpec((1, _OUT), lambda i: (0, 0)),
        ],
        out_specs=pl.BlockSpec((BT, _OUT), lambda i: (i, 0)),
        out_shape=jax.ShapeDtypeStruct((B, _OUT), jnp.float32),
        scratch_shapes=[pltpu.VMEM((BT, _NROWS), jnp.bfloat16)],
    )(X, colmax, T2, b1.reshape(1, -1), W2.T, b2.reshape(1, -1))
    return out


# speedup vs baseline: 272.0333x; 2.9025x over previous
import jax
import jax.numpy as jnp
from jax.experimental import pallas as pl
from jax.experimental.pallas import tpu as pltpu

_IN_DIM = 100
_N_BINS = 10
_EMB = 16
_HID = 64
_OUT = 64
_FPAD = 128
_NROWS = _N_BINS * _FPAD
_BT = 1024


def _fused_kernel(x_ref, embp_ref, w1s_ref, b1_ref, w2t_ref, b2_ref, o_ref,
                  cmax_scr, t2_scr, onehot_scr):
    p = pl.program_id(0)
    i = pl.program_id(1)

    @pl.when(p == 0)
    def _colmax_phase():
        part = jnp.max(jnp.abs(x_ref[...]), axis=0, keepdims=True)

        @pl.when(i == 0)
        def _():
            cmax_scr[...] = part

        @pl.when(i > 0)
        def _():
            cmax_scr[...] = jnp.maximum(cmax_scr[...], part)

    @pl.when((p == 1) & (i == 0))
    def _fold_phase():
        acc = embp_ref[:, :, 0:1] * w1s_ref[0]
        for d in range(1, _EMB):
            acc = acc + embp_ref[:, :, d:d + 1] * w1s_ref[d]
        t2_scr[...] = acc.reshape(_NROWS, _HID).astype(jnp.bfloat16)

    @pl.when(p == 1)
    def _main_phase():
        x = x_ref[...]
        d = cmax_scr[...]
        bins = jnp.clip(x / d * (_N_BINS / 2.0) + _N_BINS / 2.0,
                        0.0, _N_BINS - 1).astype(jnp.int32)
        pad = jnp.full((x.shape[0], _FPAD - _IN_DIM), -1, jnp.int32)
        binp = jnp.concatenate([bins, pad], axis=1)
        for n in range(_N_BINS):
            onehot_scr[:, n * _FPAD:(n + 1) * _FPAD] = (
                (binp == n).astype(jnp.bfloat16))
        h = jax.lax.dot(onehot_scr[...], t2_scr[...],
                        preferred_element_type=jnp.float32)
        h = jnp.maximum(h + b1_ref[...], 0.0)
        out = jax.lax.dot(h, w2t_ref[...], preferred_element_type=jnp.float32)
        o_ref[...] = out + b2_ref[...]


def kernel(X, emb, W1, b1, W2, b2):
    B, IN = X.shape
    G = B // _BT

    embp = jnp.pad(jnp.transpose(emb, (1, 0, 2)),
                   ((0, 0), (0, _FPAD - _IN_DIM), (0, 0)))
    w1s = jnp.pad(W1.T.reshape(_IN_DIM, _EMB, _HID).transpose(1, 0, 2),
                  ((0, 0), (0, _FPAD - _IN_DIM), (0, 0)))

    out = pl.pallas_call(
        _fused_kernel,
        grid=(2, G),
        in_specs=[
            pl.BlockSpec((_BT, IN), lambda p, i: (i, 0)),
            pl.BlockSpec((_N_BINS, _FPAD, _EMB), lambda p, i: (0, 0, 0)),
            pl.BlockSpec((_EMB, _FPAD, _HID), lambda p, i: (0, 0, 0)),
            pl.BlockSpec((1, _HID), lambda p, i: (0, 0)),
            pl.BlockSpec((_HID, _OUT), lambda p, i: (0, 0)),
            pl.BlockSpec((1, _OUT), lambda p, i: (0, 0)),
        ],
        out_specs=pl.BlockSpec((_BT, _OUT), lambda p, i: (i * p, 0)),
        out_shape=jax.ShapeDtypeStruct((B, _OUT), jnp.float32),
        scratch_shapes=[
            pltpu.VMEM((1, IN), jnp.float32),
            pltpu.VMEM((_NROWS, _HID), jnp.bfloat16),
            pltpu.VMEM((_BT, _NROWS), jnp.bfloat16),
        ],
    )(X, embp, w1s, b1.reshape(1, -1), W2.T, b2.reshape(1, -1))
    return out
